# baseline (speedup 1.0000x reference)
import jax
import jax.numpy as jnp
from jax import lax
from jax.experimental import pallas as pl
from jax.experimental.pallas import tpu as pltpu

_NEG = -jnp.inf


def _topk_chunk(pm, idxacc, gacc, k0, iters, n, gs, cap):
    lanes = lax.broadcasted_iota(jnp.int32, (gs, n), 1)
    kcol = lax.broadcasted_iota(jnp.int32, (gs, cap), 1)

    def body(k, carry):
        pm, idxacc, gacc = carry
        mx = jnp.max(pm, axis=1, keepdims=True)
        cand = jnp.where(pm == mx, lanes, n)
        am = jnp.min(cand, axis=1, keepdims=True)
        pm = jnp.where(lanes == am, _NEG, pm)
        sel = kcol == k
        idxacc = jnp.where(sel, am, idxacc)
        gacc = jnp.where(sel, mx, gacc)
        return pm, idxacc, gacc

    return lax.fori_loop(k0, k0 + iters, body, (pm, idxacc, gacc))


def _moe_body(x_ref, wg_ref, w1_ref, b1_ref, w2_ref, b2_ref, out_ref,
              probst_scr, pm_scr, iacc_scr, gacc_scr, idx_ec, g_ec):
    e = pl.program_id(0)
    n, d = x_ref.shape
    num_e = wg_ref.shape[1]
    cap = idx_ec.shape[1]
    gs = pm_scr.shape[0]
    ngroups = num_e // gs
    ips = cap // gs

    @pl.when(e == 0)
    def _router():
        logits_t = lax.dot_general(
            wg_ref[...], x_ref[...], (((0,), (1,)), ((), ())),
            preferred_element_type=jnp.float32)
        m = jnp.max(logits_t, axis=0, keepdims=True)
        p = jnp.exp(logits_t - m)
        probst_scr[...] = p / jnp.sum(p, axis=0, keepdims=True)

        pm = probst_scr[pl.ds(0, gs), :]
        iacc = jnp.zeros((gs, cap), jnp.int32)
        gacc = jnp.zeros((gs, cap), jnp.float32)
        _, iacc, gacc = _topk_chunk(pm, iacc, gacc, 0, cap, n, gs, cap)
        idx_ec[pl.ds(0, gs), :] = iacc
        g_ec[pl.ds(0, gs), :] = gacc

    @pl.when(e < (ngroups - 1) * gs)
    def _topk_pipe():
        g = e // gs
        j = e - g * gs
        nxt = (g + 1) * gs

        @pl.when(j == 0)
        def _seed():
            pm_scr[...] = probst_scr[pl.ds(nxt, gs), :]
            iacc_scr[...] = jnp.zeros((gs, cap), jnp.int32)
            gacc_scr[...] = jnp.zeros((gs, cap), jnp.float32)

        pm, iacc, gacc = _topk_chunk(
            pm_scr[...], iacc_scr[...], gacc_scr[...], j * ips, ips,
            n, gs, cap)
        pm_scr[...] = pm
        iacc_scr[...] = iacc
        gacc_scr[...] = gacc

        @pl.when(j == gs - 1)
        def _publish():
            idx_ec[pl.ds(nxt, gs), :] = iacc
            g_ec[pl.ds(nxt, gs), :] = gacc

    idx_row = idx_ec[pl.ds(e, 1), :]
    g_row = g_ec[pl.ds(e, 1), :]
    rows_n = lax.broadcasted_iota(jnp.int32, (n, cap), 0)
    oh = (rows_n == idx_row).astype(jnp.float32)
    disp = lax.dot_general(oh, x_ref[...], (((0,), (0,)), ((), ())),
                           preferred_element_type=jnp.float32)
    h = jnp.dot(disp, w1_ref[0], preferred_element_type=jnp.float32)
    h = jax.nn.gelu(h + b1_ref[pl.ds(e, 1), :])
    oe = jnp.dot(h, w2_ref[0], preferred_element_type=jnp.float32)
    oe = oe + b2_ref[pl.ds(e, 1), :]
    contrib = jnp.dot(oh * g_row, oe,
                      preferred_element_type=jnp.float32)

    @pl.when(e == 0)
    def _init():
        out_ref[...] = contrib

    @pl.when(e != 0)
    def _acc():
        out_ref[...] = out_ref[...] + contrib


def _moe(tokens, Wg, w1, b1, w2, b2, *, interpret=False):
    n, d = tokens.shape
    num_e = Wg.shape[1]
    f = w1.shape[2]
    cap = 2 * n // num_e
    gs = max(num_e // 8, 1)

    return pl.pallas_call(
        _moe_body,
        grid=(num_e,),
        in_specs=[
            pl.BlockSpec((n, d), lambda e: (0, 0)),
            pl.BlockSpec((d, num_e), lambda e: (0, 0)),
            pl.BlockSpec((1, d, f), lambda e: (e, 0, 0)),
            pl.BlockSpec((num_e, f), lambda e: (0, 0)),
            pl.BlockSpec((1, f, d), lambda e: (e, 0, 0)),
            pl.BlockSpec((num_e, d), lambda e: (0, 0)),
        ],
        out_specs=pl.BlockSpec((n, d), lambda e: (0, 0)),
        out_shape=jax.ShapeDtypeStruct((n, d), jnp.float32),
        scratch_shapes=[
            pltpu.VMEM((num_e, n), jnp.float32),
            pltpu.VMEM((gs, n), jnp.float32),
            pltpu.VMEM((gs, cap), jnp.int32),
            pltpu.VMEM((gs, cap), jnp.float32),
            pltpu.VMEM((num_e, cap), jnp.int32),
            pltpu.VMEM((num_e, cap), jnp.float32),
        ],
        compiler_params=pltpu.CompilerParams(
            dimension_semantics=("arbitrary",),
        ),
        interpret=interpret,
    )(tokens, Wg, w1, b1, w2, b2)


def kernel(x, Wg, w1, b1, w2, b2):
    bb, ss, dd = x.shape
    out = _moe(x.reshape(bb * ss, dd), Wg, w1, b1, w2, b2)
    return out.reshape(bb, ss, dd)


# speedup vs baseline: 1.3499x; 1.3499x over previous
import jax
import jax.numpy as jnp
from jax import lax
from jax.experimental import pallas as pl
from jax.experimental.pallas import tpu as pltpu

_RING = 3


def _moe_body(x_ref, wg_ref, b1_ref, b2_ref, w1_hbm, w2_hbm, out_ref,
              w1buf, w2buf, probs_scr, rank_scr, ltri_scr, sem1, sem2):
    e = pl.program_id(0)
    n, d = x_ref.shape
    num_e = wg_ref.shape[1]
    cap = 2 * n // num_e
    nh = n // 2
    ne2 = 2 * num_e

    def start_fetch(expert, slot):
        pltpu.make_async_copy(w1_hbm.at[expert], w1buf.at[slot],
                              sem1.at[slot]).start()
        pltpu.make_async_copy(w2_hbm.at[expert], w2buf.at[slot],
                              sem2.at[slot]).start()

    @pl.when(e == 0)
    def _prime_and_route():
        for k in range(_RING):
            start_fetch(k, k)

        tokens = x_ref[...]
        logits = jnp.dot(tokens, wg_ref[...],
                         preferred_element_type=jnp.float32)
        m = jnp.max(logits, axis=1, keepdims=True)
        p = jnp.exp(logits - m)
        p = p / jnp.sum(p, axis=1, keepdims=True)
        pf = jnp.concatenate([p[:nh, :], p[nh:, :]], axis=1)
        probs_scr[...] = pf

        ri = lax.broadcasted_iota(jnp.int32, (nh, nh), 0)
        ci = lax.broadcasted_iota(jnp.int32, (nh, nh), 1)
        ltri_scr[...] = (ci < ri).astype(jnp.bfloat16)

        pbits = lax.bitcast_convert_type(pf, jnp.int32)
        capf = jnp.float32(cap)

        def bisect(_, lohi):
            lo, hi = lohi
            mid = lo + lax.shift_right_logical(hi - lo, 1)
            ge = (pbits >= mid).astype(jnp.float32)
            cnt = jnp.sum(ge, axis=0, keepdims=True)
            cnt = cnt + pltpu.roll(cnt, num_e, 1)
            pred = jnp.clip(jnp.sign(cnt - (capf - 0.5)), 0.0, 1.0)
            predi = pred.astype(jnp.int32)
            lo = predi * mid + (1 - predi) * lo
            hi = predi * hi + (1 - predi) * mid
            return lo, hi

        lo0 = jnp.full((1, ne2), 1, jnp.int32)
        hi0 = jnp.full((1, ne2), 0x3F800001, jnp.int32)
        tau, taup1 = lax.fori_loop(0, 30, bisect, (lo0, hi0), unroll=2)

        gt = (pbits >= taup1).astype(jnp.float32)
        eq = (pbits == tau).astype(jnp.float32)
        ltri = ltri_scr[...]
        cgt = jnp.dot(ltri, gt.astype(jnp.bfloat16),
                      preferred_element_type=jnp.float32)
        ceq = jnp.dot(ltri, eq.astype(jnp.bfloat16),
                      preferred_element_type=jnp.float32)
        tot_gt = jnp.sum(gt, axis=0, keepdims=True)
        tot_eq_h = jnp.sum(eq, axis=0, keepdims=True)
        lanef = lax.broadcasted_iota(
            jnp.int32, (1, ne2), 1).astype(jnp.float32)
        ish0 = jnp.clip(jnp.sign(num_e - 0.5 - lanef), 0.0, 1.0)
        cgt = cgt + (1.0 - ish0) * pltpu.roll(tot_gt, num_e, 1)
        ceq = ceq + (1.0 - ish0) * pltpu.roll(tot_eq_h, num_e, 1)
        tot_gt_all = tot_gt + pltpu.roll(tot_gt, num_e, 1)
        eq_keep = jnp.clip(
            jnp.sign((capf - 0.5) - (tot_gt_all + ceq)), 0.0, 1.0)
        sel = gt + eq * eq_keep
        rank = gt * cgt + eq * eq_keep * (tot_gt_all + ceq)
        rank_scr[...] = sel * rank + (1.0 - sel) * capf

    slot = lax.rem(e, _RING)
    pltpu.make_async_copy(w1_hbm.at[e], w1buf.at[slot], sem1.at[slot]).wait()
    pltpu.make_async_copy(w2_hbm.at[e], w2buf.at[slot], sem2.at[slot]).wait()

    r128 = lax.broadcasted_iota(jnp.int32, (ne2, 1), 0)
    sel_h0 = (r128 == e).astype(jnp.float32)
    sel_h1 = (r128 == e + num_e).astype(jnp.float32)
    rk = rank_scr[...]
    pfv = probs_scr[...]
    rcol0 = jnp.dot(rk, sel_h0, preferred_element_type=jnp.float32)
    rcol1 = jnp.dot(rk, sel_h1, preferred_element_type=jnp.float32)
    pcol0 = jnp.dot(pfv, sel_h0, preferred_element_type=jnp.float32)
    pcol1 = jnp.dot(pfv, sel_h1, preferred_element_type=jnp.float32)
    ccols = lax.broadcasted_iota(jnp.int32, (nh, cap), 1)
    oh0 = (ccols == rcol0.astype(jnp.int32)).astype(jnp.float32)
    oh1 = (ccols == rcol1.astype(jnp.int32)).astype(jnp.float32)

    disp = (lax.dot_general(oh0, x_ref[pl.ds(0, nh), :],
                            (((0,), (0,)), ((), ())),
                            preferred_element_type=jnp.float32)
            + lax.dot_general(oh1, x_ref[pl.ds(nh, nh), :],
                              (((0,), (0,)), ((), ())),
                              preferred_element_type=jnp.float32))
    h = jnp.dot(disp, w1buf[slot], preferred_element_type=jnp.float32)
    h = jax.nn.gelu(h + b1_ref[pl.ds(e, 1), :])
    oe = jnp.dot(h, w2buf[slot], preferred_element_type=jnp.float32)
    oe = oe + b2_ref[pl.ds(e, 1), :]
    c0 = jnp.dot(oh0 * pcol0, oe, preferred_element_type=jnp.float32)
    c1 = jnp.dot(oh1 * pcol1, oe, preferred_element_type=jnp.float32)

    @pl.when(e == 0)
    def _init():
        out_ref[pl.ds(0, nh), :] = c0
        out_ref[pl.ds(nh, nh), :] = c1

    @pl.when(e != 0)
    def _acc():
        out_ref[pl.ds(0, nh), :] = out_ref[pl.ds(0, nh), :] + c0
        out_ref[pl.ds(nh, nh), :] = out_ref[pl.ds(nh, nh), :] + c1

    @pl.when(e + _RING < num_e)
    def _refill():
        start_fetch(e + _RING, slot)


def _moe(tokens, Wg, w1, b1, w2, b2, *, interpret=False):
    n, d = tokens.shape
    num_e = Wg.shape[1]
    f = w1.shape[2]

    return pl.pallas_call(
        _moe_body,
        grid=(num_e,),
        in_specs=[
            pl.BlockSpec((n, d), lambda e: (0, 0)),
            pl.BlockSpec((d, num_e), lambda e: (0, 0)),
            pl.BlockSpec((num_e, f), lambda e: (0, 0)),
            pl.BlockSpec((num_e, d), lambda e: (0, 0)),
            pl.BlockSpec(memory_space=pl.ANY),
            pl.BlockSpec(memory_space=pl.ANY),
        ],
        out_specs=pl.BlockSpec((n, d), lambda e: (0, 0)),
        out_shape=jax.ShapeDtypeStruct((n, d), jnp.float32),
        scratch_shapes=[
            pltpu.VMEM((_RING, d, f), jnp.float32),
            pltpu.VMEM((_RING, f, d), jnp.float32),
            pltpu.VMEM((n // 2, 2 * num_e), jnp.float32),
            pltpu.VMEM((n // 2, 2 * num_e), jnp.float32),
            pltpu.VMEM((n // 2, n // 2), jnp.bfloat16),
            pltpu.SemaphoreType.DMA((_RING,)),
            pltpu.SemaphoreType.DMA((_RING,)),
        ],
        compiler_params=pltpu.CompilerParams(
            dimension_semantics=("arbitrary",),
        ),
        interpret=interpret,
    )(tokens, Wg, b1, b2, w1, w2)


def kernel(x, Wg, w1, b1, w2, b2):
    bb, ss, dd = x.shape
    out = _moe(x.reshape(bb * ss, dd), Wg, w1, b1, w2, b2)
    return out.reshape(bb, ss, dd)
